# baseline (speedup 1.0000x reference)
import functools

import jax
import jax.numpy as jnp
from jax import lax
from jax.experimental import pallas as pl
from jax.experimental.pallas import tpu as pltpu
from jax.experimental.pallas import tpu_sc as plsc

B = 16384
S = 50
D = 32
L = 16
NW = 32
BPW = B // NW
C = 32
NCHUNK = BPW // C
IDS = C * S
GSLICE = 128
NG = IDS // GSLICE
GREM = IDS - NG * GSLICE


def _worker(ids_hbm, len_hbm, table_hbm, out_hbm,
            idx_a, idx_b, rows_a, rows_b, len_a, len_b, outb_a, outb_b,
            sem_a, sem_b):
    wid = lax.axis_index("s") * 2 + lax.axis_index("c")
    w_row0 = wid * BPW

    def fire(row0, idx_r, len_r, rows_r, sem):
        pltpu.sync_copy(ids_hbm.at[pl.ds(row0, C), :], idx_r)
        pltpu.sync_copy(len_hbm.at[pl.ds(row0, C)], len_r)
        for r in range(C):
            pltpu.async_copy(table_hbm.at[idx_r.at[r]],
                             rows_r.at[pl.ds(r * S, S)], sem)

    def drain(idx_r, rows_r, sem):
        pltpu.make_async_copy(table_hbm.at[pl.ds(0, IDS)], rows_r, sem).wait()

    def compute(row0, len_r, rows_r, outb_r):
        for g in range(C // L):
            lenv = len_r[pl.ds(g * L, L)]

            def row_body(r, _):
                lenb = lax.gather(
                    lenv, jnp.full((L, 1), r, jnp.int32),
                    lax.GatherDimensionNumbers(
                        offset_dims=(), collapsed_slice_dims=(0,),
                        start_index_map=(0,)),
                    slice_sizes=(1,),
                    mode=lax.GatherScatterMode.PROMISE_IN_BOUNDS)
                zero_reg = lax.shift_right_logical(
                    lax.broadcasted_iota(jnp.int32, (L,), 0), 4)
                lenb = lenb + zero_reg
                base = (g * L + r) * S

                def s_body(s, acc):
                    a0, a1 = acc
                    m = jnp.full((L,), s, jnp.int32) < lenb
                    v0 = rows_r[base + s, pl.ds(0, L)]
                    v1 = rows_r[base + s, pl.ds(L, L)]
                    zero = jnp.zeros((L,), jnp.float32)
                    return (a0 + jnp.where(m, v0, zero),
                            a1 + jnp.where(m, v1, zero))

                a0, a1 = lax.fori_loop(
                    0, S, s_body,
                    (jnp.zeros((L,), jnp.float32),
                     jnp.zeros((L,), jnp.float32)))
                denom = jnp.maximum(lenb, 1).astype(jnp.float32)
                outb_r[pl.ds((g * L + r) * D, L)] = a0 / denom
                outb_r[pl.ds((g * L + r) * D + L, L)] = a1 / denom
                return 0

            lax.fori_loop(0, L, row_body, 0)
        pltpu.sync_copy(outb_r, out_hbm.at[pl.ds(row0 * D, C * D)])

    fire(w_row0, idx_a, len_a, rows_a, sem_a)

    def outer(i, _):
        g0row = w_row0 + (2 * i) * C
        fire(g0row + C, idx_b, len_b, rows_b, sem_b)
        drain(idx_a, rows_a, sem_a)
        compute(g0row, len_a, rows_a, outb_a)

        @pl.when(i < NCHUNK // 2 - 1)
        def _():
            fire(g0row + 2 * C, idx_a, len_a, rows_a, sem_a)

        drain(idx_b, rows_b, sem_b)
        compute(g0row + C, len_b, rows_b, outb_b)
        return 0

    lax.fori_loop(0, NCHUNK // 2, outer, 0)


@functools.partial(
    pl.kernel,
    mesh=plsc.VectorSubcoreMesh(core_axis_name="c", subcore_axis_name="s"),
    out_type=jax.ShapeDtypeStruct((B * D,), jnp.float32),
    compiler_params=pltpu.CompilerParams(use_tc_tiling_on_sc=False),
    scratch_types=[
        pltpu.VMEM((C, S), jnp.int32), pltpu.VMEM((C, S), jnp.int32),
        pltpu.VMEM((IDS, D), jnp.float32), pltpu.VMEM((IDS, D), jnp.float32),
        pltpu.VMEM((C,), jnp.int32), pltpu.VMEM((C,), jnp.int32),
        pltpu.VMEM((C * D,), jnp.float32), pltpu.VMEM((C * D,), jnp.float32),
        pltpu.SemaphoreType.DMA, pltpu.SemaphoreType.DMA,
    ],
)
def _embed_kernel(ids_hbm, len_hbm, table_hbm, out_hbm, *scratch):
    _worker(ids_hbm, len_hbm, table_hbm, out_hbm, *scratch)


def kernel(input_ids, lengths, table):
    out = _embed_kernel(input_ids.astype(jnp.int32), lengths.astype(jnp.int32),
                        table)
    return out.reshape(B, D)


# speedup vs baseline: 1.0015x; 1.0015x over previous
import functools

import jax
import jax.numpy as jnp
from jax import lax
from jax.experimental import pallas as pl
from jax.experimental.pallas import tpu as pltpu
from jax.experimental.pallas import tpu_sc as plsc

VOCAB = 1000000
B = 16384
S = 50
D = 32
L = 16
NW = 32
BPW = B // NW
C = 32
NCHUNK = BPW // C
IDS = C * S
GSLICE = 128
NG = IDS // GSLICE
GREM = IDS - NG * GSLICE


def _worker(ids_hbm, len_hbm, table_hbm, out_hbm,
            idx_a, idx_b, rows_a, rows_b, len_a, len_b, outb_a, outb_b,
            sem_a, sem_b):
    wid = lax.axis_index("s") * 2 + lax.axis_index("c")
    w_row0 = wid * BPW

    def fire(row0, idx_r, len_r, rows_r, sem):
        pltpu.sync_copy(ids_hbm.at[pl.ds(row0, C), :], idx_r)
        pltpu.sync_copy(len_hbm.at[pl.ds(row0, C)], len_r)
        for r in range(C):
            pltpu.async_copy(table_hbm.at[idx_r.at[r]],
                             rows_r.at[pl.ds(r * S, S)], sem)

    def drain(idx_r, rows_r, sem):
        pltpu.make_async_copy(table_hbm.at[pl.ds(0, IDS)], rows_r, sem).wait()

    def compute(row0, len_r, rows_r, outb_r):
        for g in range(C // L):
            lenv = len_r[pl.ds(g * L, L)]

            def row_body(r, _):
                lenb = lax.gather(
                    lenv, jnp.full((L, 1), r, jnp.int32),
                    lax.GatherDimensionNumbers(
                        offset_dims=(), collapsed_slice_dims=(0,),
                        start_index_map=(0,)),
                    slice_sizes=(1,),
                    mode=lax.GatherScatterMode.PROMISE_IN_BOUNDS)
                zero_reg = lax.shift_right_logical(
                    lax.broadcasted_iota(jnp.int32, (L,), 0), 4)
                lenb = lenb + zero_reg
                base = (g * L + r) * S

                def s_body(s, acc):
                    a0, a1 = acc
                    m = jnp.full((L,), s, jnp.int32) < lenb
                    v0 = rows_r[base + s, pl.ds(0, L)]
                    v1 = rows_r[base + s, pl.ds(L, L)]
                    zero = jnp.zeros((L,), jnp.float32)
                    return (a0 + jnp.where(m, v0, zero),
                            a1 + jnp.where(m, v1, zero))

                a0, a1 = lax.fori_loop(
                    0, S, s_body,
                    (jnp.zeros((L,), jnp.float32),
                     jnp.zeros((L,), jnp.float32)))
                denom = jnp.maximum(lenb, 1).astype(jnp.float32)
                outb_r[pl.ds((g * L + r) * D, L)] = a0 / denom
                outb_r[pl.ds((g * L + r) * D + L, L)] = a1 / denom
                return 0

            lax.fori_loop(0, L, row_body, 0)
        pltpu.sync_copy(outb_r, out_hbm.at[pl.ds(row0 * D, C * D)])

    fire(w_row0, idx_a, len_a, rows_a, sem_a)

    def outer(i, _):
        g0row = w_row0 + (2 * i) * C
        fire(g0row + C, idx_b, len_b, rows_b, sem_b)
        drain(idx_a, rows_a, sem_a)
        compute(g0row, len_a, rows_a, outb_a)

        @pl.when(i < NCHUNK // 2 - 1)
        def _():
            fire(g0row + 2 * C, idx_a, len_a, rows_a, sem_a)

        drain(idx_b, rows_b, sem_b)
        compute(g0row + C, len_b, rows_b, outb_b)
        return 0

    lax.fori_loop(0, NCHUNK // 2, outer, 0)


@functools.partial(
    pl.kernel,
    mesh=plsc.VectorSubcoreMesh(core_axis_name="c", subcore_axis_name="s"),
    out_type=jax.ShapeDtypeStruct((B * D,), jnp.float32),
    compiler_params=pltpu.CompilerParams(use_tc_tiling_on_sc=False),
    scratch_types=[
        pltpu.VMEM((C, S), jnp.int32), pltpu.VMEM((C, S), jnp.int32),
        pltpu.VMEM((IDS, D), jnp.float32), pltpu.VMEM((IDS, D), jnp.float32),
        pltpu.VMEM((C,), jnp.int32), pltpu.VMEM((C,), jnp.int32),
        pltpu.VMEM((C * D,), jnp.float32), pltpu.VMEM((C * D,), jnp.float32),
        pltpu.SemaphoreType.DMA, pltpu.SemaphoreType.DMA,
    ],
)
def _embed_kernel(ids_hbm, len_hbm, table_hbm, out_hbm, *scratch):
    _worker(ids_hbm, len_hbm, table_hbm, out_hbm, *scratch)


TCB = 2048


def _tc_transpose_body(tt_ref, out_ref):
    t = jnp.transpose(tt_ref[...])
    t3 = t.reshape(TCB // 4, 4, D)
    o = jnp.concatenate([t3[:, q, :] for q in range(4)], axis=1)
    out_ref[...] = o.reshape(TCB * D)


_tc_transpose = pl.pallas_call(
    _tc_transpose_body,
    grid=(pl.cdiv(VOCAB, TCB),),
    in_specs=[pl.BlockSpec((D, TCB), lambda i: (0, i))],
    out_specs=pl.BlockSpec((TCB * D,), lambda i: (i,)),
    out_shape=jax.ShapeDtypeStruct((VOCAB * D,), jnp.float32),
)


def kernel(input_ids, lengths, table):
    tbl_lin = _tc_transpose(table.T).reshape(VOCAB, D)
    out = _embed_kernel(input_ids.astype(jnp.int32), lengths.astype(jnp.int32),
                        tbl_lin)
    return out.reshape(B, D)


# speedup vs baseline: 1.3142x; 1.3122x over previous
import functools

import jax
import jax.numpy as jnp
from jax import lax
from jax.experimental import pallas as pl
from jax.experimental.pallas import tpu as pltpu
from jax.experimental.pallas import tpu_sc as plsc

VOCAB = 1000000
B = 16384
S = 50
D = 32
L = 16
NW = 32
BPW = B // NW
C = 32
NCHUNK = BPW // C
IDS = C * S
GSLICE = 128
NG = IDS // GSLICE
GREM = IDS - NG * GSLICE


def _worker(ids_hbm, len_hbm, table_hbm, out_hbm,
            idx_a, idx_b, rows_a, rows_b, len_a, len_b, outb_a, outb_b,
            sem_a, sem_b):
    wid = lax.axis_index("s") * 2 + lax.axis_index("c")
    w_row0 = wid * BPW

    def fire(row0, idx_r, len_r, rows_r, sem):
        pltpu.sync_copy(ids_hbm.at[pl.ds(row0 * S, IDS)], idx_r)
        pltpu.sync_copy(len_hbm.at[pl.ds(row0, C)], len_r)

        def xform(k, _):
            v = idx_r[pl.ds(k * L, L)]
            idx_r[pl.ds(k * L, L)] = (
                (v & -512) + ((v & 127) << 2) + ((v >> 7) & 3))
            return 0

        lax.fori_loop(0, IDS // L, xform, 0)
        for j in range(NG):
            pltpu.async_copy(
                table_hbm.at[idx_r.at[pl.ds(j * GSLICE, GSLICE)]],
                rows_r.at[pl.ds(j * GSLICE, GSLICE)], sem)
        pltpu.async_copy(
            table_hbm.at[idx_r.at[pl.ds(NG * GSLICE, GREM)]],
            rows_r.at[pl.ds(NG * GSLICE, GREM)], sem)

    def drain(idx_r, rows_r, sem):
        pltpu.make_async_copy(table_hbm.at[pl.ds(0, IDS)], rows_r, sem).wait()

    def compute(row0, len_r, rows_r, outb_r):
        for g in range(C // L):
            lenv = len_r[pl.ds(g * L, L)]

            def row_body(r, _):
                lenb = lax.gather(
                    lenv, jnp.full((L, 1), r, jnp.int32),
                    lax.GatherDimensionNumbers(
                        offset_dims=(), collapsed_slice_dims=(0,),
                        start_index_map=(0,)),
                    slice_sizes=(1,),
                    mode=lax.GatherScatterMode.PROMISE_IN_BOUNDS)
                zero_reg = lax.shift_right_logical(
                    lax.broadcasted_iota(jnp.int32, (L,), 0), 4)
                lenb = lenb + zero_reg
                base = (g * L + r) * S

                def s_body(s, acc):
                    a0, a1 = acc
                    m = jnp.full((L,), s, jnp.int32) < lenb
                    v0 = rows_r[base + s, pl.ds(0, L)]
                    v1 = rows_r[base + s, pl.ds(L, L)]
                    zero = jnp.zeros((L,), jnp.float32)
                    return (a0 + jnp.where(m, v0, zero),
                            a1 + jnp.where(m, v1, zero))

                a0, a1 = lax.fori_loop(
                    0, S, s_body,
                    (jnp.zeros((L,), jnp.float32),
                     jnp.zeros((L,), jnp.float32)))
                denom = jnp.maximum(lenb, 1).astype(jnp.float32)
                outb_r[pl.ds((g * L + r) * D, L)] = a0 / denom
                outb_r[pl.ds((g * L + r) * D + L, L)] = a1 / denom
                return 0

            lax.fori_loop(0, L, row_body, 0)
        pltpu.sync_copy(outb_r, out_hbm.at[pl.ds(row0 * D, C * D)])

    fire(w_row0, idx_a, len_a, rows_a, sem_a)

    def outer(i, _):
        g0row = w_row0 + (2 * i) * C
        fire(g0row + C, idx_b, len_b, rows_b, sem_b)
        drain(idx_a, rows_a, sem_a)
        compute(g0row, len_a, rows_a, outb_a)

        @pl.when(i < NCHUNK // 2 - 1)
        def _():
            fire(g0row + 2 * C, idx_a, len_a, rows_a, sem_a)

        drain(idx_b, rows_b, sem_b)
        compute(g0row + C, len_b, rows_b, outb_b)
        return 0

    lax.fori_loop(0, NCHUNK // 2, outer, 0)


@functools.partial(
    pl.kernel,
    mesh=plsc.VectorSubcoreMesh(core_axis_name="c", subcore_axis_name="s"),
    out_type=jax.ShapeDtypeStruct((B * D,), jnp.float32),
    compiler_params=pltpu.CompilerParams(use_tc_tiling_on_sc=False),
    scratch_types=[
        pltpu.VMEM((IDS,), jnp.int32), pltpu.VMEM((IDS,), jnp.int32),
        pltpu.VMEM((IDS, D), jnp.float32), pltpu.VMEM((IDS, D), jnp.float32),
        pltpu.VMEM((C,), jnp.int32), pltpu.VMEM((C,), jnp.int32),
        pltpu.VMEM((C * D,), jnp.float32), pltpu.VMEM((C * D,), jnp.float32),
        pltpu.SemaphoreType.DMA, pltpu.SemaphoreType.DMA,
    ],
)
def _embed_kernel(ids_hbm, len_hbm, table_hbm, out_hbm, *scratch):
    _worker(ids_hbm, len_hbm, table_hbm, out_hbm, *scratch)


TCB = 2048


def _tc_transpose_body(tt_ref, out_ref):
    blk = tt_ref[...]
    outs = []
    for sg in range(TCB // 512):
        stacked = jnp.concatenate(
            [blk[:, sg * 512 + q * 128:sg * 512 + (q + 1) * 128]
             for q in range(4)], axis=0)
        outs.append(jnp.transpose(stacked))
    o = jnp.concatenate(outs, axis=0)
    out_ref[...] = o.reshape(TCB * D)


NTB = pl.cdiv(VOCAB, TCB)
VOCAB_PAD = NTB * TCB

_tc_transpose = pl.pallas_call(
    _tc_transpose_body,
    grid=(NTB,),
    in_specs=[pl.BlockSpec((D, TCB), lambda i: (0, i))],
    out_specs=pl.BlockSpec((TCB * D,), lambda i: (i,)),
    out_shape=jax.ShapeDtypeStruct((VOCAB_PAD * D,), jnp.float32),
)


def kernel(input_ids, lengths, table):
    tbl_lin = _tc_transpose(table.T).reshape(VOCAB_PAD, D)
    out = _embed_kernel(input_ids.reshape(-1).astype(jnp.int32),
                        lengths.astype(jnp.int32), tbl_lin)
    return out.reshape(B, D)


# speedup vs baseline: 2.0969x; 1.5955x over previous
import functools

import jax
import jax.numpy as jnp
from jax import lax
from jax.experimental import pallas as pl
from jax.experimental.pallas import tpu as pltpu
from jax.experimental.pallas import tpu_sc as plsc

VOCAB = 1000000
B = 16384
S = 50
D = 32
L = 16
NW = 32
BPW = B // NW
C = 32
NCHUNK = BPW // C
IDS = C * S
GSLICE = 128
NG = IDS // GSLICE
GREM = IDS - NG * GSLICE


def _worker(ids_hbm, len_hbm, table_hbm, out_hbm,
            idx_a, idx_b, rows_a, rows_b, len_a, len_b, outb_a, outb_b,
            sem_a, sem_b):
    wid = lax.axis_index("s") * 2 + lax.axis_index("c")
    w_row0 = wid * BPW

    def fire(row0, idx_r, len_r, rows_r, sem):
        pltpu.sync_copy(ids_hbm.at[pl.ds(row0 * S, IDS)], idx_r)
        pltpu.sync_copy(len_hbm.at[pl.ds(row0, C)], len_r)

        def xform(k, _):
            v = idx_r[pl.ds(k * L, L)]
            idx_r[pl.ds(k * L, L)] = (
                (v & -512) + ((v & 127) << 2) + ((v >> 7) & 3))
            return 0

        lax.fori_loop(0, IDS // L, xform, 0)
        for j in range(NG):
            pltpu.async_copy(
                table_hbm.at[idx_r.at[pl.ds(j * GSLICE, GSLICE)]],
                rows_r.at[pl.ds(j * GSLICE, GSLICE)], sem)
        pltpu.async_copy(
            table_hbm.at[idx_r.at[pl.ds(NG * GSLICE, GREM)]],
            rows_r.at[pl.ds(NG * GSLICE, GREM)], sem)

    def drain(idx_r, rows_r, sem):
        pltpu.make_async_copy(table_hbm.at[pl.ds(0, IDS)], rows_r, sem).wait()

    def compute(row0, len_r, rows_r, outb_r):
        for g in range(C // L):
            lenv = len_r[pl.ds(g * L, L)]

            def row_body(r, _):
                lenb = lax.gather(
                    lenv, jnp.full((L, 1), r, jnp.int32),
                    lax.GatherDimensionNumbers(
                        offset_dims=(), collapsed_slice_dims=(0,),
                        start_index_map=(0,)),
                    slice_sizes=(1,),
                    mode=lax.GatherScatterMode.PROMISE_IN_BOUNDS)
                zero_reg = lax.shift_right_logical(
                    lax.broadcasted_iota(jnp.int32, (L,), 0), 4)
                lenb = lenb + zero_reg
                base = (g * L + r) * S

                def s_body(s, acc):
                    a0, a1 = acc
                    m = jnp.full((L,), s, jnp.int32) < lenb
                    v0 = rows_r[base + s, pl.ds(0, L)]
                    v1 = rows_r[base + s, pl.ds(L, L)]
                    zero = jnp.zeros((L,), jnp.float32)
                    return (a0 + jnp.where(m, v0, zero),
                            a1 + jnp.where(m, v1, zero))

                a0, a1 = lax.fori_loop(
                    0, S, s_body,
                    (jnp.zeros((L,), jnp.float32),
                     jnp.zeros((L,), jnp.float32)))
                denom = jnp.maximum(lenb, 1).astype(jnp.float32)
                outb_r[pl.ds((g * L + r) * D, L)] = a0 / denom
                outb_r[pl.ds((g * L + r) * D + L, L)] = a1 / denom
                return 0

            lax.fori_loop(0, L, row_body, 0)
        pltpu.sync_copy(outb_r, out_hbm.at[pl.ds(row0 * D, C * D)])

    fire(w_row0, idx_a, len_a, rows_a, sem_a)

    def outer(i, _):
        g0row = w_row0 + (2 * i) * C
        fire(g0row + C, idx_b, len_b, rows_b, sem_b)
        drain(idx_a, rows_a, sem_a)
        compute(g0row, len_a, rows_a, outb_a)

        @pl.when(i < NCHUNK // 2 - 1)
        def _():
            fire(g0row + 2 * C, idx_a, len_a, rows_a, sem_a)

        drain(idx_b, rows_b, sem_b)
        compute(g0row + C, len_b, rows_b, outb_b)
        return 0

    lax.fori_loop(0, NCHUNK // 2, outer, 0)


@functools.partial(
    pl.kernel,
    mesh=plsc.VectorSubcoreMesh(core_axis_name="c", subcore_axis_name="s"),
    out_type=jax.ShapeDtypeStruct((B * D,), jnp.float32),
    compiler_params=pltpu.CompilerParams(use_tc_tiling_on_sc=False),
    scratch_types=[
        pltpu.VMEM((IDS,), jnp.int32), pltpu.VMEM((IDS,), jnp.int32),
        pltpu.VMEM((IDS, D), jnp.float32), pltpu.VMEM((IDS, D), jnp.float32),
        pltpu.VMEM((C,), jnp.int32), pltpu.VMEM((C,), jnp.int32),
        pltpu.VMEM((C * D,), jnp.float32), pltpu.VMEM((C * D,), jnp.float32),
        pltpu.SemaphoreType.DMA, pltpu.SemaphoreType.DMA,
    ],
)
def _embed_kernel(ids_hbm, len_hbm, table_hbm, out_hbm, *scratch):
    _worker(ids_hbm, len_hbm, table_hbm, out_hbm, *scratch)


TCB = 8192


def _tc_transpose_body(tt_ref, out_ref):
    blk = tt_ref[...]
    outs = []
    for sg in range(TCB // 512):
        stacked = jnp.concatenate(
            [blk[:, sg * 512 + q * 128:sg * 512 + (q + 1) * 128]
             for q in range(4)], axis=0)
        outs.append(jnp.transpose(stacked))
    o = jnp.concatenate(outs, axis=0)
    out_ref[...] = o.reshape(TCB * D)


NTB = pl.cdiv(VOCAB, TCB)
VOCAB_PAD = NTB * TCB

_tc_transpose = pl.pallas_call(
    _tc_transpose_body,
    grid=(NTB,),
    in_specs=[pl.BlockSpec((D, TCB), lambda i: (0, i))],
    out_specs=pl.BlockSpec((TCB * D,), lambda i: (i,)),
    out_shape=jax.ShapeDtypeStruct((VOCAB_PAD * D,), jnp.float32),
)


def kernel(input_ids, lengths, table):
    tbl_lin = _tc_transpose(table.T).reshape(VOCAB_PAD, D)
    out = _embed_kernel(input_ids.reshape(-1).astype(jnp.int32),
                        lengths.astype(jnp.int32), tbl_lin)
    return out.reshape(B, D)


# speedup vs baseline: 2.5172x; 1.2005x over previous
import functools

import jax
import jax.numpy as jnp
from jax import lax
from jax.experimental import pallas as pl
from jax.experimental.pallas import tpu as pltpu
from jax.experimental.pallas import tpu_sc as plsc

VOCAB = 1000000
B = 16384
S = 50
D = 32
L = 16
NW = 32
BPW = B // NW
C = 32
NCHUNK = BPW // C
IDS = C * S
GSLICE = 128
NG = IDS // GSLICE
GREM = IDS - NG * GSLICE


def _worker(ids_hbm, len_hbm, table_hbm, out_hbm,
            idx_a, idx_b, rows_a, rows_b, len_a, len_b, outb_a, outb_b,
            sem_a, sem_b):
    wid = lax.axis_index("s") * 2 + lax.axis_index("c")
    w_row0 = wid * BPW

    def fire(row0, idx_r, len_r, rows_r, sem):
        pltpu.sync_copy(ids_hbm.at[pl.ds(row0 * S, IDS)], idx_r)
        pltpu.sync_copy(len_hbm.at[pl.ds(row0, C)], len_r)

        def xform(k, _):
            v = idx_r[pl.ds(k * L, L)]
            idx_r[pl.ds(k * L, L)] = (
                (v & -512) + ((v & 127) << 2) + ((v >> 7) & 3))
            return 0

        lax.fori_loop(0, IDS // L, xform, 0)
        for j in range(NG):
            pltpu.async_copy(
                table_hbm.at[idx_r.at[pl.ds(j * GSLICE, GSLICE)]],
                rows_r.at[pl.ds(j * GSLICE, GSLICE)], sem)
        pltpu.async_copy(
            table_hbm.at[idx_r.at[pl.ds(NG * GSLICE, GREM)]],
            rows_r.at[pl.ds(NG * GSLICE, GREM)], sem)

    def drain(idx_r, rows_r, sem):
        pltpu.make_async_copy(table_hbm.at[pl.ds(0, IDS)], rows_r, sem).wait()

    def compute(row0, len_r, rows_r, outb_r):
        for g in range(C // L):
            lenv = len_r[pl.ds(g * L, L)]

            def row_body(r, _):
                lenb = lax.gather(
                    lenv, jnp.full((L, 1), r, jnp.int32),
                    lax.GatherDimensionNumbers(
                        offset_dims=(), collapsed_slice_dims=(0,),
                        start_index_map=(0,)),
                    slice_sizes=(1,),
                    mode=lax.GatherScatterMode.PROMISE_IN_BOUNDS)
                zero_reg = lax.shift_right_logical(
                    lax.broadcasted_iota(jnp.int32, (L,), 0), 4)
                lenb = lenb + zero_reg
                base = (g * L + r) * S

                def s_body(s, acc):
                    a0, a1 = acc
                    m = jnp.full((L,), s, jnp.int32) < lenb
                    v0 = rows_r[base + s, pl.ds(0, L)]
                    v1 = rows_r[base + s, pl.ds(L, L)]
                    zero = jnp.zeros((L,), jnp.float32)
                    return (a0 + jnp.where(m, v0, zero),
                            a1 + jnp.where(m, v1, zero))

                a0, a1 = lax.fori_loop(
                    0, S, s_body,
                    (jnp.zeros((L,), jnp.float32),
                     jnp.zeros((L,), jnp.float32)))
                denom = jnp.maximum(lenb, 1).astype(jnp.float32)
                outb_r[pl.ds((g * L + r) * D, L)] = a0 / denom
                outb_r[pl.ds((g * L + r) * D + L, L)] = a1 / denom
                return 0

            lax.fori_loop(0, L, row_body, 0)
        pltpu.sync_copy(outb_r, out_hbm.at[pl.ds(row0 * D, C * D)])

    fire(w_row0, idx_a, len_a, rows_a, sem_a)

    def outer(i, _):
        g0row = w_row0 + (2 * i) * C
        fire(g0row + C, idx_b, len_b, rows_b, sem_b)
        drain(idx_a, rows_a, sem_a)
        compute(g0row, len_a, rows_a, outb_a)

        @pl.when(i < NCHUNK // 2 - 1)
        def _():
            fire(g0row + 2 * C, idx_a, len_a, rows_a, sem_a)

        drain(idx_b, rows_b, sem_b)
        compute(g0row + C, len_b, rows_b, outb_b)
        return 0

    lax.fori_loop(0, NCHUNK // 2, outer, 0)


@functools.partial(
    pl.kernel,
    mesh=plsc.VectorSubcoreMesh(core_axis_name="c", subcore_axis_name="s"),
    out_type=jax.ShapeDtypeStruct((B * D,), jnp.float32),
    compiler_params=pltpu.CompilerParams(use_tc_tiling_on_sc=False),
    scratch_types=[
        pltpu.VMEM((IDS,), jnp.int32), pltpu.VMEM((IDS,), jnp.int32),
        pltpu.VMEM((IDS, D), jnp.float32), pltpu.VMEM((IDS, D), jnp.float32),
        pltpu.VMEM((C,), jnp.int32), pltpu.VMEM((C,), jnp.int32),
        pltpu.VMEM((C * D,), jnp.float32), pltpu.VMEM((C * D,), jnp.float32),
        pltpu.SemaphoreType.DMA, pltpu.SemaphoreType.DMA,
    ],
)
def _embed_kernel(ids_hbm, len_hbm, table_hbm, out_hbm, *scratch):
    _worker(ids_hbm, len_hbm, table_hbm, out_hbm, *scratch)


TCB = 32768


def _tc_transpose_body(tt_ref, out_ref):
    blk = tt_ref[...]
    outs = []
    for sg in range(TCB // 512):
        stacked = jnp.concatenate(
            [blk[:, sg * 512 + q * 128:sg * 512 + (q + 1) * 128]
             for q in range(4)], axis=0)
        outs.append(jnp.transpose(stacked))
    o = jnp.concatenate(outs, axis=0)
    out_ref[...] = o.reshape(TCB * D)


NTB = pl.cdiv(VOCAB, TCB)
VOCAB_PAD = NTB * TCB

_tc_transpose = pl.pallas_call(
    _tc_transpose_body,
    grid=(NTB,),
    in_specs=[pl.BlockSpec((D, TCB), lambda i: (0, i))],
    out_specs=pl.BlockSpec((TCB * D,), lambda i: (i,)),
    out_shape=jax.ShapeDtypeStruct((VOCAB_PAD * D,), jnp.float32),
)


def kernel(input_ids, lengths, table):
    tbl_lin = _tc_transpose(table.T).reshape(VOCAB_PAD, D)
    out = _embed_kernel(input_ids.reshape(-1).astype(jnp.int32),
                        lengths.astype(jnp.int32), tbl_lin)
    return out.reshape(B, D)


# speedup vs baseline: 2.5347x; 1.0070x over previous
import functools

import jax
import jax.numpy as jnp
from jax import lax
from jax.experimental import pallas as pl
from jax.experimental.pallas import tpu as pltpu
from jax.experimental.pallas import tpu_sc as plsc

VOCAB = 1000000
B = 16384
S = 50
D = 32
L = 16
NW = 32
BPW = B // NW
C = 32
NCHUNK = BPW // C
IDS = C * S
GSLICE = 128
NG = IDS // GSLICE
GREM = IDS - NG * GSLICE


def _worker(ids_hbm, len_hbm, table_hbm, out_hbm,
            idx_a, idx_b, rows_a, rows_b, len_a, len_b, outb_a, outb_b,
            sem_a, sem_b):
    wid = lax.axis_index("s") * 2 + lax.axis_index("c")
    w_row0 = wid * BPW

    def fire(row0, idx_r, len_r, rows_r, sem):
        pltpu.sync_copy(ids_hbm.at[pl.ds(row0 * S, IDS)], idx_r)
        pltpu.sync_copy(len_hbm.at[pl.ds(row0, C)], len_r)

        def xform(k, _):
            v = idx_r[pl.ds(k * L, L)]
            idx_r[pl.ds(k * L, L)] = (
                (v & -512) + ((v & 127) << 2) + ((v >> 7) & 3))
            return 0

        lax.fori_loop(0, IDS // L, xform, 0)
        for j in range(NG):
            pltpu.async_copy(
                table_hbm.at[idx_r.at[pl.ds(j * GSLICE, GSLICE)]],
                rows_r.at[pl.ds(j * GSLICE, GSLICE)], sem)
        pltpu.async_copy(
            table_hbm.at[idx_r.at[pl.ds(NG * GSLICE, GREM)]],
            rows_r.at[pl.ds(NG * GSLICE, GREM)], sem)

    def drain(idx_r, rows_r, sem):
        pltpu.make_async_copy(table_hbm.at[pl.ds(0, IDS)], rows_r, sem).wait()

    def compute(row0, len_r, rows_r, outb_r):
        for g in range(C // L):
            lenv = len_r[pl.ds(g * L, L)]

            def row_body(r, _):
                lenb = lax.gather(
                    lenv, jnp.full((L, 1), r, jnp.int32),
                    lax.GatherDimensionNumbers(
                        offset_dims=(), collapsed_slice_dims=(0,),
                        start_index_map=(0,)),
                    slice_sizes=(1,),
                    mode=lax.GatherScatterMode.PROMISE_IN_BOUNDS)
                zero_reg = lax.shift_right_logical(
                    lax.broadcasted_iota(jnp.int32, (L,), 0), 4)
                lenb = lenb + zero_reg
                base = (g * L + r) * S

                def s_body(s, acc):
                    a0, a1 = acc
                    m = jnp.full((L,), s, jnp.int32) < lenb
                    v0 = rows_r[base + s, pl.ds(0, L)]
                    v1 = rows_r[base + s, pl.ds(L, L)]
                    zero = jnp.zeros((L,), jnp.float32)
                    return (a0 + jnp.where(m, v0, zero),
                            a1 + jnp.where(m, v1, zero))

                a0, a1 = lax.fori_loop(
                    0, S, s_body,
                    (jnp.zeros((L,), jnp.float32),
                     jnp.zeros((L,), jnp.float32)))
                denom = jnp.maximum(lenb, 1).astype(jnp.float32)
                outb_r[pl.ds((g * L + r) * D, L)] = a0 / denom
                outb_r[pl.ds((g * L + r) * D + L, L)] = a1 / denom
                return 0

            lax.fori_loop(0, L, row_body, 0)
        pltpu.sync_copy(outb_r, out_hbm.at[pl.ds(row0 * D, C * D)])

    fire(w_row0, idx_a, len_a, rows_a, sem_a)

    def outer(i, _):
        g0row = w_row0 + (2 * i) * C
        fire(g0row + C, idx_b, len_b, rows_b, sem_b)
        drain(idx_a, rows_a, sem_a)
        compute(g0row, len_a, rows_a, outb_a)

        @pl.when(i < NCHUNK // 2 - 1)
        def _():
            fire(g0row + 2 * C, idx_a, len_a, rows_a, sem_a)

        drain(idx_b, rows_b, sem_b)
        compute(g0row + C, len_b, rows_b, outb_b)
        return 0

    lax.fori_loop(0, NCHUNK // 2, outer, 0)


@functools.partial(
    pl.kernel,
    mesh=plsc.VectorSubcoreMesh(core_axis_name="c", subcore_axis_name="s"),
    out_type=jax.ShapeDtypeStruct((B * D,), jnp.float32),
    compiler_params=pltpu.CompilerParams(use_tc_tiling_on_sc=False),
    scratch_types=[
        pltpu.VMEM((IDS,), jnp.int32), pltpu.VMEM((IDS,), jnp.int32),
        pltpu.VMEM((IDS, D), jnp.float32), pltpu.VMEM((IDS, D), jnp.float32),
        pltpu.VMEM((C,), jnp.int32), pltpu.VMEM((C,), jnp.int32),
        pltpu.VMEM((C * D,), jnp.float32), pltpu.VMEM((C * D,), jnp.float32),
        pltpu.SemaphoreType.DMA, pltpu.SemaphoreType.DMA,
    ],
)
def _embed_kernel(ids_hbm, len_hbm, table_hbm, out_hbm, *scratch):
    _worker(ids_hbm, len_hbm, table_hbm, out_hbm, *scratch)


TCB = 65536


def _tc_transpose_body(tt_ref, out_ref):
    blk = tt_ref[...]
    outs = []
    for sg in range(TCB // 512):
        stacked = jnp.concatenate(
            [blk[:, sg * 512 + q * 128:sg * 512 + (q + 1) * 128]
             for q in range(4)], axis=0)
        outs.append(jnp.transpose(stacked))
    o = jnp.concatenate(outs, axis=0)
    out_ref[...] = o.reshape(TCB * D)


NTB = pl.cdiv(VOCAB, TCB)
VOCAB_PAD = NTB * TCB

_tc_transpose = pl.pallas_call(
    _tc_transpose_body,
    grid=(NTB,),
    in_specs=[pl.BlockSpec((D, TCB), lambda i: (0, i))],
    out_specs=pl.BlockSpec((TCB * D,), lambda i: (i,)),
    out_shape=jax.ShapeDtypeStruct((VOCAB_PAD * D,), jnp.float32),
)


def kernel(input_ids, lengths, table):
    tbl_lin = _tc_transpose(table.T).reshape(VOCAB_PAD, D)
    out = _embed_kernel(input_ids.reshape(-1).astype(jnp.int32),
                        lengths.astype(jnp.int32), tbl_lin)
    return out.reshape(B, D)


# speedup vs baseline: 2.6527x; 1.0465x over previous
import functools

import jax
import jax.numpy as jnp
from jax import lax
from jax.experimental import pallas as pl
from jax.experimental.pallas import tpu as pltpu
from jax.experimental.pallas import tpu_sc as plsc

VOCAB = 1000000
B = 16384
S = 50
D = 32
L = 16
NW = 32
BPW = B // NW
C = 32
NCHUNK = BPW // C
IDS = C * S
GSLICE = 128
NG = IDS // GSLICE
GREM = IDS - NG * GSLICE


def _worker(ids_hbm, len_hbm, table_hbm, out_hbm,
            ids_all, len_all, rows_a, rows_b, outb_a, outb_b,
            sem_a, sem_b):
    wid = lax.axis_index("s") * 2 + lax.axis_index("c")
    w_row0 = wid * BPW

    pltpu.sync_copy(ids_hbm.at[pl.ds(w_row0 * S, BPW * S)], ids_all)
    pltpu.sync_copy(len_hbm.at[pl.ds(w_row0, BPW)], len_all)

    def xform(k, _):
        v = ids_all[pl.ds(k * L, L)]
        ids_all[pl.ds(k * L, L)] = (
            (v & -512) + ((v & 127) << 2) + ((v >> 7) & 3))
        return 0

    lax.fori_loop(0, BPW * S // L, xform, 0)

    def fire(g, rows_r, sem):
        base = g * IDS
        for j in range(NG):
            pltpu.async_copy(
                table_hbm.at[ids_all.at[pl.ds(base + j * GSLICE, GSLICE)]],
                rows_r.at[pl.ds(j * GSLICE, GSLICE)], sem)
        pltpu.async_copy(
            table_hbm.at[ids_all.at[pl.ds(base + NG * GSLICE, GREM)]],
            rows_r.at[pl.ds(NG * GSLICE, GREM)], sem)

    def drain(rows_r, sem):
        pltpu.make_async_copy(table_hbm.at[pl.ds(0, IDS)], rows_r, sem).wait()

    def compute(g, rows_r, outb_r):
        row0 = w_row0 + g * C
        for grp in range(C // L):
            lenv = len_all[pl.ds(g * C + grp * L, L)]

            def row_body(r, _):
                lenb = lax.gather(
                    lenv, jnp.full((L, 1), r, jnp.int32),
                    lax.GatherDimensionNumbers(
                        offset_dims=(), collapsed_slice_dims=(0,),
                        start_index_map=(0,)),
                    slice_sizes=(1,),
                    mode=lax.GatherScatterMode.PROMISE_IN_BOUNDS)
                zero_reg = lax.shift_right_logical(
                    lax.broadcasted_iota(jnp.int32, (L,), 0), 4)
                lenb = lenb + zero_reg
                base = (grp * L + r) * S

                def s_body(s, acc):
                    a0, a1 = acc
                    m = jnp.full((L,), s, jnp.int32) < lenb
                    v0 = rows_r[base + s, pl.ds(0, L)]
                    v1 = rows_r[base + s, pl.ds(L, L)]
                    zero = jnp.zeros((L,), jnp.float32)
                    return (a0 + jnp.where(m, v0, zero),
                            a1 + jnp.where(m, v1, zero))

                a0, a1 = lax.fori_loop(
                    0, S, s_body,
                    (jnp.zeros((L,), jnp.float32),
                     jnp.zeros((L,), jnp.float32)))
                denom = jnp.maximum(lenb, 1).astype(jnp.float32)
                outb_r[pl.ds((grp * L + r) * D, L)] = a0 / denom
                outb_r[pl.ds((grp * L + r) * D + L, L)] = a1 / denom
                return 0

            lax.fori_loop(0, L, row_body, 0)
        pltpu.sync_copy(outb_r, out_hbm.at[pl.ds(row0 * D, C * D)])

    fire(0, rows_a, sem_a)

    def outer(i, _):
        g0 = 2 * i
        fire(g0 + 1, rows_b, sem_b)
        drain(rows_a, sem_a)
        compute(g0, rows_a, outb_a)

        @pl.when(i < NCHUNK // 2 - 1)
        def _():
            fire(g0 + 2, rows_a, sem_a)

        drain(rows_b, sem_b)
        compute(g0 + 1, rows_b, outb_b)
        return 0

    lax.fori_loop(0, NCHUNK // 2, outer, 0)


@functools.partial(
    pl.kernel,
    mesh=plsc.VectorSubcoreMesh(core_axis_name="c", subcore_axis_name="s"),
    out_type=jax.ShapeDtypeStruct((B * D,), jnp.float32),
    compiler_params=pltpu.CompilerParams(use_tc_tiling_on_sc=False),
    scratch_types=[
        pltpu.VMEM((BPW * S,), jnp.int32), pltpu.VMEM((BPW,), jnp.int32),
        pltpu.VMEM((IDS, D), jnp.float32), pltpu.VMEM((IDS, D), jnp.float32),
        pltpu.VMEM((C * D,), jnp.float32), pltpu.VMEM((C * D,), jnp.float32),
        pltpu.SemaphoreType.DMA, pltpu.SemaphoreType.DMA,
    ],
)
def _embed_kernel(ids_hbm, len_hbm, table_hbm, out_hbm, *scratch):
    _worker(ids_hbm, len_hbm, table_hbm, out_hbm, *scratch)


TCB = 65536


def _tc_transpose_body(tt_ref, out_ref):
    blk = tt_ref[...]
    outs = []
    for sg in range(TCB // 512):
        stacked = jnp.concatenate(
            [blk[:, sg * 512 + q * 128:sg * 512 + (q + 1) * 128]
             for q in range(4)], axis=0)
        outs.append(jnp.transpose(stacked))
    o = jnp.concatenate(outs, axis=0)
    out_ref[...] = o.reshape(TCB * D)


NTB = pl.cdiv(VOCAB, TCB)
VOCAB_PAD = NTB * TCB

_tc_transpose = pl.pallas_call(
    _tc_transpose_body,
    grid=(NTB,),
    in_specs=[pl.BlockSpec((D, TCB), lambda i: (0, i))],
    out_specs=pl.BlockSpec((TCB * D,), lambda i: (i,)),
    out_shape=jax.ShapeDtypeStruct((VOCAB_PAD * D,), jnp.float32),
)


def kernel(input_ids, lengths, table):
    tbl_lin = _tc_transpose(table.T).reshape(VOCAB_PAD, D)
    out = _embed_kernel(input_ids.reshape(-1).astype(jnp.int32),
                        lengths.astype(jnp.int32), tbl_lin)
    return out.reshape(B, D)
